# Initial kernel scaffold; baseline (speedup 1.0000x reference)
import jax
import jax.numpy as jnp
from jax.experimental import pallas as pl


def kernel(feat, edge_index, W, b):
    raise NotImplementedError("write your pallas kernel here")


# speedup vs baseline: 9.1101x; 9.1101x over previous
import functools

import jax
import jax.numpy as jnp
from jax import lax
from jax.experimental import pallas as pl
from jax.experimental.pallas import tpu as pltpu
from jax.experimental.pallas import tpu_sc as plsc

NC = 2
NS = 16
NW = NC * NS
CHUNK = 128
IGRP = 16
ROW_BLK = 256


def _agg_body(featsrc_hbm, srcidx_hbm, dstidx_hbm, zeros_hbm, out_hbm,
              src_v, dst_v, buf0, buf1, acc, sem0, sem1, *, cpw, n_pad, d):
    cid = lax.axis_index("c")
    sid = lax.axis_index("s")
    wid = cid * NS + sid
    rps = n_pad // NS

    pltpu.sync_copy(zeros_hbm.at[pl.ds(sid * rps, rps)],
                    acc.at[pl.ds(sid * rps, rps)])
    plsc.subcore_barrier()

    def group(g, carry):
        pltpu.sync_copy(srcidx_hbm.at[wid, pl.ds(g * IGRP, IGRP)], src_v)
        pltpu.sync_copy(dstidx_hbm.at[wid, pl.ds(g * IGRP, IGRP)], dst_v)

        def step(t, carry2):
            j0 = 2 * t
            j1 = 2 * t + 1
            c0 = pltpu.async_copy(featsrc_hbm.at[src_v.at[j0]], buf0, sem0)
            c1 = pltpu.async_copy(featsrc_hbm.at[src_v.at[j1]], buf1, sem1)
            c0.wait()
            pltpu.sync_copy(buf0, acc.at[dst_v.at[j0]], add=True)
            c1.wait()
            pltpu.sync_copy(buf1, acc.at[dst_v.at[j1]], add=True)
            return carry2

        lax.fori_loop(0, IGRP // 2, step, 0)
        return carry

    lax.fori_loop(0, cpw // IGRP, group, 0)
    plsc.subcore_barrier()
    pltpu.sync_copy(acc.at[pl.ds(sid * rps, rps)],
                    out_hbm.at[cid, pl.ds(sid * rps, rps)])


def _count_body(srcidx_hbm, dstidx_hbm, zeros_hbm, out_hbm,
                src_v, dst_v, ones_v, scnt, dcnt, *, cpw, n_pad):
    cid = lax.axis_index("c")
    sid = lax.axis_index("s")
    wid = cid * NS + sid
    rps = n_pad // NS

    pltpu.sync_copy(zeros_hbm.at[pl.ds(sid * rps, rps)],
                    scnt.at[pl.ds(sid * rps, rps)])
    pltpu.sync_copy(zeros_hbm.at[pl.ds(sid * rps, rps)],
                    dcnt.at[pl.ds(sid * rps, rps)])
    for k in range(CHUNK // 16):
        ones_v[pl.ds(k * 16, 16)] = jnp.full((16,), 1.0, jnp.float32)
    pltpu.sync_copy(srcidx_hbm.at[wid], src_v)
    pltpu.sync_copy(dstidx_hbm.at[wid], dst_v)
    plsc.subcore_barrier()

    def step(j, carry):
        pltpu.sync_copy(ones_v, scnt.at[src_v.at[j]], add=True)
        pltpu.sync_copy(ones_v, dcnt.at[dst_v.at[j]], add=True)
        return carry

    lax.fori_loop(0, cpw, step, 0)
    plsc.subcore_barrier()
    pltpu.sync_copy(scnt.at[pl.ds(sid * rps, rps)],
                    out_hbm.at[cid, 0, pl.ds(sid * rps, rps)])
    pltpu.sync_copy(dcnt.at[pl.ds(sid * rps, rps)],
                    out_hbm.at[cid, 1, pl.ds(sid * rps, rps)])


def _scale_body(cnt_ref, feat_ref, out_ref):
    src_cnt = cnt_ref[0, 0, :] + cnt_ref[1, 0, :]
    ns = lax.rsqrt(jnp.maximum(src_cnt, 1.0))
    out_ref[...] = feat_ref[...] * ns[:, None]


def _final_body(agg_ref, cnt_ref, feat_ref, wt_ref, b_ref, out_ref):
    a = agg_ref[0] + agg_ref[1]
    r = jnp.dot(a, wt_ref[...], preferred_element_type=jnp.float32)
    r = r + b_ref[...]
    dst_cnt = cnt_ref[0, 1, :] + cnt_ref[1, 1, :]
    nd = lax.rsqrt(jnp.maximum(dst_cnt, 1.0))
    out_ref[...] = r * nd[:, None] + feat_ref[...]


def kernel(feat, edge_index, W, b):
    n, d = feat.shape
    e = edge_index.shape[1]

    n_pad = -(-(n + 1) // ROW_BLK) * ROW_BLK
    epw_quant = IGRP * CHUNK
    e_pad = -(-e // (NW * epw_quant)) * (NW * epw_quant)
    cpw = e_pad // (NW * CHUNK)

    pad_e = e_pad - e
    fill = (n + jnp.arange(pad_e, dtype=jnp.int32) % (n_pad - n)).astype(jnp.int32)
    src_r = jnp.concatenate([edge_index[0].astype(jnp.int32), fill])
    src_r = src_r.reshape(NW, cpw, CHUNK)
    dst_r = jnp.concatenate([edge_index[1].astype(jnp.int32), fill])
    dst_r = dst_r.reshape(NW, cpw, CHUNK)

    feat_p = jnp.concatenate(
        [feat, jnp.zeros((n_pad - n, d), jnp.float32)], axis=0)
    zeros1 = jnp.zeros((n_pad,), jnp.float32)
    zeros2 = jnp.zeros((n_pad, d), jnp.float32)

    mesh = plsc.VectorSubcoreMesh(core_axis_name="c", subcore_axis_name="s")

    count_k = pl.kernel(
        functools.partial(_count_body, cpw=cpw, n_pad=n_pad),
        out_type=jax.ShapeDtypeStruct((NC, 2, n_pad), jnp.float32),
        mesh=mesh,
        scratch_types=[
            pltpu.VMEM((cpw, CHUNK), jnp.int32),
            pltpu.VMEM((cpw, CHUNK), jnp.int32),
            pltpu.VMEM((CHUNK,), jnp.float32),
            pltpu.VMEM_SHARED((n_pad,), jnp.float32),
            pltpu.VMEM_SHARED((n_pad,), jnp.float32),
        ],
    )
    cnt = count_k(src_r, dst_r, zeros1)

    grid = n_pad // ROW_BLK
    feat_src = pl.pallas_call(
        _scale_body,
        grid=(grid,),
        in_specs=[
            pl.BlockSpec((NC, 2, ROW_BLK), lambda i: (0, 0, i)),
            pl.BlockSpec((ROW_BLK, d), lambda i: (i, 0)),
        ],
        out_specs=pl.BlockSpec((ROW_BLK, d), lambda i: (i, 0)),
        out_shape=jax.ShapeDtypeStruct((n_pad, d), jnp.float32),
    )(cnt, feat_p)

    agg_k = pl.kernel(
        functools.partial(_agg_body, cpw=cpw, n_pad=n_pad, d=d),
        out_type=jax.ShapeDtypeStruct((NC, n_pad, d), jnp.float32),
        mesh=mesh,
        scratch_types=[
            pltpu.VMEM((IGRP, CHUNK), jnp.int32),
            pltpu.VMEM((IGRP, CHUNK), jnp.int32),
            pltpu.VMEM((CHUNK, d), jnp.float32),
            pltpu.VMEM((CHUNK, d), jnp.float32),
            pltpu.VMEM_SHARED((n_pad, d), jnp.float32),
            pltpu.SemaphoreType.DMA,
            pltpu.SemaphoreType.DMA,
        ],
    )
    agg = agg_k(feat_src, src_r, dst_r, zeros2)

    out_p = pl.pallas_call(
        _final_body,
        grid=(grid,),
        in_specs=[
            pl.BlockSpec((NC, ROW_BLK, d), lambda i: (0, i, 0)),
            pl.BlockSpec((NC, 2, ROW_BLK), lambda i: (0, 0, i)),
            pl.BlockSpec((ROW_BLK, d), lambda i: (i, 0)),
            pl.BlockSpec((d, d), lambda i: (0, 0)),
            pl.BlockSpec((1, d), lambda i: (0, 0)),
        ],
        out_specs=pl.BlockSpec((ROW_BLK, d), lambda i: (i, 0)),
        out_shape=jax.ShapeDtypeStruct((n_pad, d), jnp.float32),
    )(agg, cnt, feat_p, W.T, b.reshape(1, d))

    return out_p[:n]


# speedup vs baseline: 9.6771x; 1.0622x over previous
import functools

import jax
import jax.numpy as jnp
from jax import lax
from jax.experimental import pallas as pl
from jax.experimental.pallas import tpu as pltpu
from jax.experimental.pallas import tpu_sc as plsc

NC = 2
NS = 16
NW = NC * NS
CHUNK = 128
IGRP = 16
ROW_BLK = 256


def _agg_body(featsrc_hbm, srcidx_hbm, dstidx_hbm, out_hbm,
              src_v, dst_v, buf0, buf1, acc, gsem0, gsem1, ssem0, ssem1,
              *, cpw, n_pad, d):
    cid = lax.axis_index("c")
    sid = lax.axis_index("s")
    wid = cid * NS + sid
    rps = n_pad // NS

    def zrow(i, carry):
        for k in range(d // 16):
            buf0[i, pl.ds(k * 16, 16)] = jnp.zeros((16,), jnp.float32)
        return carry

    lax.fori_loop(0, CHUNK, zrow, 0)
    for r in range(rps // CHUNK):
        pltpu.sync_copy(buf0, acc.at[pl.ds(sid * rps + r * CHUNK, CHUNK)])
    plsc.subcore_barrier()

    def group(g, carry):
        pltpu.sync_copy(srcidx_hbm.at[wid, pl.ds(g * IGRP, IGRP)], src_v)
        pltpu.sync_copy(dstidx_hbm.at[wid, pl.ds(g * IGRP, IGRP)], dst_v)
        nstep = IGRP // 2

        g0 = pltpu.async_copy(featsrc_hbm.at[src_v.at[0]], buf0, gsem0)
        g1 = pltpu.async_copy(featsrc_hbm.at[src_v.at[1]], buf1, gsem1)

        def step(t, carry2):
            j0 = 2 * t
            j1 = 2 * t + 1
            pltpu.make_async_copy(featsrc_hbm.at[src_v.at[j0]], buf0,
                                  gsem0).wait()
            s0 = pltpu.async_copy(buf0, acc.at[dst_v.at[j0]], ssem0,
                                  add=True)
            pltpu.make_async_copy(featsrc_hbm.at[src_v.at[j1]], buf1,
                                  gsem1).wait()
            s1 = pltpu.async_copy(buf1, acc.at[dst_v.at[j1]], ssem1,
                                  add=True)
            s0.wait()

            @pl.when(t < nstep - 1)
            def _():
                pltpu.async_copy(featsrc_hbm.at[src_v.at[j0 + 2]], buf0,
                                 gsem0)

            s1.wait()

            @pl.when(t < nstep - 1)
            def _():
                pltpu.async_copy(featsrc_hbm.at[src_v.at[j1 + 2]], buf1,
                                 gsem1)

            return carry2

        lax.fori_loop(0, nstep, step, 0)
        return carry

    lax.fori_loop(0, cpw // IGRP, group, 0)
    plsc.subcore_barrier()
    pltpu.sync_copy(acc.at[pl.ds(sid * rps, rps)],
                    out_hbm.at[cid, pl.ds(sid * rps, rps)])


def _count_body(srcidx_hbm, dstidx_hbm, zeros_hbm, out_hbm,
                src_v, dst_v, ones_v, scnt, dcnt, *, cpw, n_pad):
    cid = lax.axis_index("c")
    sid = lax.axis_index("s")
    wid = cid * NS + sid
    rps = n_pad // NS

    pltpu.sync_copy(zeros_hbm.at[pl.ds(sid * rps, rps)],
                    scnt.at[pl.ds(sid * rps, rps)])
    pltpu.sync_copy(zeros_hbm.at[pl.ds(sid * rps, rps)],
                    dcnt.at[pl.ds(sid * rps, rps)])
    for k in range(CHUNK // 16):
        ones_v[pl.ds(k * 16, 16)] = jnp.full((16,), 1.0, jnp.float32)
    pltpu.sync_copy(srcidx_hbm.at[wid], src_v)
    pltpu.sync_copy(dstidx_hbm.at[wid], dst_v)
    plsc.subcore_barrier()

    def step(j, carry):
        pltpu.sync_copy(ones_v, scnt.at[src_v.at[j]], add=True)
        pltpu.sync_copy(ones_v, dcnt.at[dst_v.at[j]], add=True)
        return carry

    lax.fori_loop(0, cpw, step, 0)
    plsc.subcore_barrier()
    pltpu.sync_copy(scnt.at[pl.ds(sid * rps, rps)],
                    out_hbm.at[cid, 0, pl.ds(sid * rps, rps)])
    pltpu.sync_copy(dcnt.at[pl.ds(sid * rps, rps)],
                    out_hbm.at[cid, 1, pl.ds(sid * rps, rps)])


def _scale_body(cnt_ref, feat_ref, out_ref, *, n):
    src_cnt = cnt_ref[0, 0, :] + cnt_ref[1, 0, :]
    ns = lax.rsqrt(jnp.maximum(src_cnt, 1.0))
    rows = (pl.program_id(0) * ROW_BLK
            + lax.broadcasted_iota(jnp.int32, (ROW_BLK, 1), 0))
    out_ref[...] = jnp.where(rows < n, feat_ref[...] * ns[:, None], 0.0)


def _final_body(agg_ref, cnt_ref, feat_ref, wt_ref, b_ref, out_ref):
    a = agg_ref[0] + agg_ref[1]
    r = jnp.dot(a, wt_ref[...], preferred_element_type=jnp.float32)
    r = r + b_ref[...]
    dst_cnt = cnt_ref[0, 1, :] + cnt_ref[1, 1, :]
    nd = lax.rsqrt(jnp.maximum(dst_cnt, 1.0))
    out_ref[...] = r * nd[:, None] + feat_ref[...]


def kernel(feat, edge_index, W, b):
    n, d = feat.shape
    e = edge_index.shape[1]

    n_pad = -(-(n + 1) // ROW_BLK) * ROW_BLK
    epw_quant = IGRP * CHUNK
    e_pad = -(-e // (NW * epw_quant)) * (NW * epw_quant)
    cpw = e_pad // (NW * CHUNK)

    pad_e = e_pad - e
    fill = (n + jnp.arange(pad_e, dtype=jnp.int32) % (n_pad - n)).astype(jnp.int32)
    src_r = jnp.concatenate([edge_index[0].astype(jnp.int32), fill])
    src_r = src_r.reshape(NW, cpw, CHUNK)
    dst_r = jnp.concatenate([edge_index[1].astype(jnp.int32), fill])
    dst_r = dst_r.reshape(NW, cpw, CHUNK)

    zeros1 = jnp.zeros((n_pad,), jnp.float32)

    mesh = plsc.VectorSubcoreMesh(core_axis_name="c", subcore_axis_name="s")

    count_k = pl.kernel(
        functools.partial(_count_body, cpw=cpw, n_pad=n_pad),
        out_type=jax.ShapeDtypeStruct((NC, 2, n_pad), jnp.float32),
        mesh=mesh,
        scratch_types=[
            pltpu.VMEM((cpw, CHUNK), jnp.int32),
            pltpu.VMEM((cpw, CHUNK), jnp.int32),
            pltpu.VMEM((CHUNK,), jnp.float32),
            pltpu.VMEM_SHARED((n_pad,), jnp.float32),
            pltpu.VMEM_SHARED((n_pad,), jnp.float32),
        ],
    )
    cnt = count_k(src_r, dst_r, zeros1)

    grid = n_pad // ROW_BLK
    feat_src = pl.pallas_call(
        functools.partial(_scale_body, n=n),
        grid=(grid,),
        in_specs=[
            pl.BlockSpec((NC, 2, ROW_BLK), lambda i: (0, 0, i)),
            pl.BlockSpec((ROW_BLK, d), lambda i: (i, 0)),
        ],
        out_specs=pl.BlockSpec((ROW_BLK, d), lambda i: (i, 0)),
        out_shape=jax.ShapeDtypeStruct((n_pad, d), jnp.float32),
    )(cnt, feat)

    agg_k = pl.kernel(
        functools.partial(_agg_body, cpw=cpw, n_pad=n_pad, d=d),
        out_type=jax.ShapeDtypeStruct((NC, n_pad, d), jnp.float32),
        mesh=mesh,
        scratch_types=[
            pltpu.VMEM((IGRP, CHUNK), jnp.int32),
            pltpu.VMEM((IGRP, CHUNK), jnp.int32),
            pltpu.VMEM((CHUNK, d), jnp.float32),
            pltpu.VMEM((CHUNK, d), jnp.float32),
            pltpu.VMEM_SHARED((n_pad, d), jnp.float32),
            pltpu.SemaphoreType.DMA,
            pltpu.SemaphoreType.DMA,
            pltpu.SemaphoreType.DMA,
            pltpu.SemaphoreType.DMA,
        ],
    )
    agg = agg_k(feat_src, src_r, dst_r)

    return pl.pallas_call(
        _final_body,
        grid=(grid,),
        in_specs=[
            pl.BlockSpec((NC, ROW_BLK, d), lambda i: (0, i, 0)),
            pl.BlockSpec((NC, 2, ROW_BLK), lambda i: (0, 0, i)),
            pl.BlockSpec((ROW_BLK, d), lambda i: (i, 0)),
            pl.BlockSpec((d, d), lambda i: (0, 0)),
            pl.BlockSpec((1, d), lambda i: (0, 0)),
        ],
        out_specs=pl.BlockSpec((ROW_BLK, d), lambda i: (i, 0)),
        out_shape=jax.ShapeDtypeStruct((n, d), jnp.float32),
    )(agg, cnt, feat, W.T, b.reshape(1, d))


# speedup vs baseline: 12.0953x; 1.2499x over previous
import functools

import jax
import jax.numpy as jnp
from jax import lax
from jax.experimental import pallas as pl
from jax.experimental.pallas import tpu as pltpu
from jax.experimental.pallas import tpu_sc as plsc

NC = 2
NS = 16
NW = NC * NS
CHUNK = 64
IGRP = 16
NBUF = 4
ROW_BLK = 1024


def _count_body(edges_hbm, out_hbm, src_v, dst_v, ones_v, scnt, dcnt, sems,
                semd, *, ngrp, n_pad):
    cid = lax.axis_index("c")
    sid = lax.axis_index("s")
    wid = cid * NS + sid
    rps = n_pad // NS
    zblk = 2 * CHUNK

    for k in range(zblk // 16):
        ones_v[pl.ds(k * 16, 16)] = jnp.zeros((16,), jnp.float32)

    def zcopy(r, carry):
        pltpu.sync_copy(ones_v, scnt.at[pl.ds(sid * rps + r * zblk, zblk)])
        pltpu.sync_copy(ones_v, dcnt.at[pl.ds(sid * rps + r * zblk, zblk)])
        return carry

    lax.fori_loop(0, rps // zblk, zcopy, 0)
    for k in range(CHUNK // 16):
        ones_v[pl.ds(k * 16, 16)] = jnp.full((16,), 1.0, jnp.float32)
    plsc.subcore_barrier()
    ones = ones_v.at[pl.ds(0, CHUNK)]

    def group(g, carry):
        pltpu.sync_copy(edges_hbm.at[0, wid, pl.ds(g * IGRP, IGRP)], src_v)
        pltpu.sync_copy(edges_hbm.at[1, wid, pl.ds(g * IGRP, IGRP)], dst_v)
        for j in range(IGRP):
            pltpu.async_copy(ones, scnt.at[src_v.at[j]], sems, add=True)
            pltpu.async_copy(ones, dcnt.at[dst_v.at[j]], semd, add=True)
        for j in range(IGRP):
            pltpu.make_async_copy(ones, scnt.at[src_v.at[j]], sems).wait()
            pltpu.make_async_copy(ones, dcnt.at[dst_v.at[j]], semd).wait()
        return carry

    lax.fori_loop(0, ngrp, group, 0)
    plsc.subcore_barrier()
    pltpu.sync_copy(scnt.at[pl.ds(sid * rps, rps)],
                    out_hbm.at[cid, 0, pl.ds(sid * rps, rps)])
    pltpu.sync_copy(dcnt.at[pl.ds(sid * rps, rps)],
                    out_hbm.at[cid, 1, pl.ds(sid * rps, rps)])


def _agg_body(featsrc_hbm, edges_hbm, out_hbm,
              src_v, dst_v, b0, b1, b2, b3, acc,
              g0, g1, g2, g3, s0, s1, s2, s3, *, ngrp, n_pad, d):
    cid = lax.axis_index("c")
    sid = lax.axis_index("s")
    wid = cid * NS + sid
    rps = n_pad // NS
    bufs = (b0, b1, b2, b3)
    gsem = (g0, g1, g2, g3)
    ssem = (s0, s1, s2, s3)

    def zrow(i, carry):
        for k in range(d // 16):
            b0[i, pl.ds(k * 16, 16)] = jnp.zeros((16,), jnp.float32)
        return carry

    lax.fori_loop(0, CHUNK, zrow, 0)
    for r in range(rps // CHUNK):
        pltpu.sync_copy(b0, acc.at[pl.ds(sid * rps + r * CHUNK, CHUNK)])
    plsc.subcore_barrier()

    def group(g, carry):
        pltpu.sync_copy(edges_hbm.at[0, wid, pl.ds(g * IGRP, IGRP)], src_v)
        pltpu.sync_copy(edges_hbm.at[1, wid, pl.ds(g * IGRP, IGRP)], dst_v)
        for k in range(NBUF):
            pltpu.async_copy(featsrc_hbm.at[src_v.at[k]], bufs[k], gsem[k])
        nround = IGRP // NBUF
        for r in range(nround):
            for k in range(NBUF):
                j = r * NBUF + k
                pltpu.make_async_copy(featsrc_hbm.at[src_v.at[j]], bufs[k],
                                      gsem[k]).wait()
                pltpu.async_copy(bufs[k], acc.at[dst_v.at[j]], ssem[k],
                                 add=True)
            for k in range(NBUF):
                j = r * NBUF + k
                pltpu.make_async_copy(bufs[k], acc.at[dst_v.at[j]],
                                      ssem[k]).wait()
                if r < nround - 1:
                    jn = (r + 1) * NBUF + k
                    pltpu.async_copy(featsrc_hbm.at[src_v.at[jn]], bufs[k],
                                     gsem[k])
        return carry

    lax.fori_loop(0, ngrp, group, 0)
    plsc.subcore_barrier()
    pltpu.sync_copy(acc.at[pl.ds(sid * rps, rps)],
                    out_hbm.at[cid, pl.ds(sid * rps, rps)])


def _scale_body(cnt_ref, feat_ref, wt_ref, out_ref, *, n):
    src_cnt = cnt_ref[0, 0, :] + cnt_ref[1, 0, :]
    ns = lax.rsqrt(jnp.maximum(src_cnt, 1.0))
    rows = (pl.program_id(0) * ROW_BLK
            + lax.broadcasted_iota(jnp.int32, (ROW_BLK, 1), 0))
    y = jnp.dot(feat_ref[...], wt_ref[...],
                preferred_element_type=jnp.float32)
    out_ref[...] = jnp.where(rows < n, y * ns[:, None], 0.0)


def _final_body(agg_ref, cnt_ref, feat_ref, b_ref, out_ref):
    a = agg_ref[0] + agg_ref[1]
    dst_cnt = cnt_ref[0, 1, :] + cnt_ref[1, 1, :]
    nd = lax.rsqrt(jnp.maximum(dst_cnt, 1.0))[:, None]
    out_ref[...] = (a + b_ref[...]) * nd + feat_ref[...]


def kernel(feat, edge_index, W, b):
    n, d = feat.shape
    e = edge_index.shape[1]

    n_pad = -(-(n + 1) // ROW_BLK) * ROW_BLK
    epq = NW * IGRP * CHUNK
    e_pad = -(-e // epq) * epq
    ngrp = e_pad // epq

    pad_e = e_pad - e
    fill = (n + jnp.arange(pad_e, dtype=jnp.int32) % (n_pad - n))
    fill = fill.astype(jnp.int32)
    edges = jnp.concatenate(
        [edge_index.astype(jnp.int32), jnp.stack([fill, fill])], axis=1)
    edges = edges.reshape(2, NW, ngrp * IGRP, CHUNK)

    mesh = plsc.VectorSubcoreMesh(core_axis_name="c", subcore_axis_name="s")

    count_k = pl.kernel(
        functools.partial(_count_body, ngrp=ngrp, n_pad=n_pad),
        out_type=jax.ShapeDtypeStruct((NC, 2, n_pad), jnp.float32),
        mesh=mesh,
        scratch_types=[
            pltpu.VMEM((IGRP, CHUNK), jnp.int32),
            pltpu.VMEM((IGRP, CHUNK), jnp.int32),
            pltpu.VMEM((2 * CHUNK,), jnp.float32),
            pltpu.VMEM_SHARED((n_pad,), jnp.float32),
            pltpu.VMEM_SHARED((n_pad,), jnp.float32),
            pltpu.SemaphoreType.DMA,
            pltpu.SemaphoreType.DMA,
        ],
    )
    cnt = count_k(edges)

    grid = n_pad // ROW_BLK
    feat_src = pl.pallas_call(
        functools.partial(_scale_body, n=n),
        grid=(grid,),
        in_specs=[
            pl.BlockSpec((NC, 2, ROW_BLK), lambda i: (0, 0, i)),
            pl.BlockSpec((ROW_BLK, d), lambda i: (i, 0)),
            pl.BlockSpec((d, d), lambda i: (0, 0)),
        ],
        out_specs=pl.BlockSpec((ROW_BLK, d), lambda i: (i, 0)),
        out_shape=jax.ShapeDtypeStruct((n_pad, d), jnp.float32),
    )(cnt, feat, W.T)

    agg_k = pl.kernel(
        functools.partial(_agg_body, ngrp=ngrp, n_pad=n_pad, d=d),
        out_type=jax.ShapeDtypeStruct((NC, n_pad, d), jnp.float32),
        mesh=mesh,
        scratch_types=[
            pltpu.VMEM((IGRP, CHUNK), jnp.int32),
            pltpu.VMEM((IGRP, CHUNK), jnp.int32),
            pltpu.VMEM((CHUNK, d), jnp.float32),
            pltpu.VMEM((CHUNK, d), jnp.float32),
            pltpu.VMEM((CHUNK, d), jnp.float32),
            pltpu.VMEM((CHUNK, d), jnp.float32),
            pltpu.VMEM_SHARED((n_pad, d), jnp.float32),
            pltpu.SemaphoreType.DMA,
            pltpu.SemaphoreType.DMA,
            pltpu.SemaphoreType.DMA,
            pltpu.SemaphoreType.DMA,
            pltpu.SemaphoreType.DMA,
            pltpu.SemaphoreType.DMA,
            pltpu.SemaphoreType.DMA,
            pltpu.SemaphoreType.DMA,
        ],
    )
    agg = agg_k(feat_src, edges)

    return pl.pallas_call(
        _final_body,
        grid=(grid,),
        in_specs=[
            pl.BlockSpec((NC, ROW_BLK, d), lambda i: (0, i, 0)),
            pl.BlockSpec((NC, 2, ROW_BLK), lambda i: (0, 0, i)),
            pl.BlockSpec((ROW_BLK, d), lambda i: (i, 0)),
            pl.BlockSpec((1, d), lambda i: (0, 0)),
        ],
        out_specs=pl.BlockSpec((ROW_BLK, d), lambda i: (i, 0)),
        out_shape=jax.ShapeDtypeStruct((n, d), jnp.float32),
    )(agg, cnt, feat, b.reshape(1, d))
